# baseline (speedup 1.0000x reference)
import jax
import jax.numpy as jnp
from jax import lax
from jax.experimental import pallas as pl
from jax.experimental.pallas import tpu as pltpu
from jax.experimental.pallas import tpu_sc as plsc

_DIM = 1024
_SCALE = _DIM ** (-0.5)

_NC = 2
_NS = 16
_NW = _NC * _NS
_LANES = 16

_ROWS = 8192
_ROWS_PER_W = _ROWS // _NW
_CHUNK_ROWS = 8
_NCHUNK = _ROWS_PER_W // _CHUNK_ROWS
_NBUF = 12
_AHEAD = _NBUF - 2
_VECS_PER_ROW = _DIM // _LANES


def _sc_scale_copy(src_hbm, out_hbm, *scratch):
    bufs = scratch[:_NBUF]
    rsems = scratch[_NBUF:2 * _NBUF]
    wsems = scratch[2 * _NBUF:]
    wid = lax.axis_index("s") * _NC + lax.axis_index("c")
    base = wid * _ROWS_PER_W

    def rows(k):
        return pl.ds(base + k * _CHUNK_ROWS, _CHUNK_ROWS)

    rd = [None] * _NCHUNK
    wr = [None] * _NCHUNK
    for k in range(_AHEAD):
        rd[k] = pltpu.async_copy(src_hbm.at[rows(k)], bufs[k % _NBUF], rsems[k % _NBUF])
    for k in range(_NCHUNK):
        par = k % _NBUF
        rd[k].wait()
        buf = bufs[par]

        @plsc.parallel_loop(0, _CHUNK_ROWS * _VECS_PER_ROW, 1, unroll=8)
        def _scale(i):
            r = i // _VECS_PER_ROW
            c = (i % _VECS_PER_ROW) * _LANES
            buf[r, pl.ds(c, _LANES)] = buf[r, pl.ds(c, _LANES)] * _SCALE

        wr[k] = pltpu.async_copy(buf, out_hbm.at[rows(k)], wsems[par])
        nk = k + _AHEAD
        if nk < _NCHUNK:
            if nk >= _NBUF:
                wr[nk - _NBUF].wait()
            rd[nk] = pltpu.async_copy(
                src_hbm.at[rows(nk)], bufs[nk % _NBUF], rsems[nk % _NBUF])
    for k in range(_NCHUNK - _NBUF, _NCHUNK):
        wr[k].wait()


def kernel(x, embed):
    seq_len = x.shape[1]
    mesh = plsc.VectorSubcoreMesh(
        core_axis_name="c", subcore_axis_name="s",
        num_cores=_NC, num_subcores=_NS,
    )
    run = pl.kernel(
        _sc_scale_copy,
        out_type=jax.ShapeDtypeStruct((seq_len, _DIM), jnp.float32),
        mesh=mesh,
        scratch_types=(
            [pltpu.VMEM((_CHUNK_ROWS, _DIM), jnp.float32)] * _NBUF
            + [pltpu.SemaphoreType.DMA] * (2 * _NBUF)
        ),
    )
    return run(embed)


# speedup vs baseline: 1.0222x; 1.0222x over previous
import jax
import jax.numpy as jnp
from jax import lax
from jax.experimental import pallas as pl
from jax.experimental.pallas import tpu as pltpu
from jax.experimental.pallas import tpu_sc as plsc

_DIM = 1024
_SCALE = _DIM ** (-0.5)

_NC = 2
_NS = 16
_NW = _NC * _NS
_LANES = 16

_ROWS = 8192
_ROWS_PER_W = _ROWS // _NW
_CHUNK_ROWS = 16
_NCHUNK = _ROWS_PER_W // _CHUNK_ROWS
_NBUF = 7
_AHEAD = _NBUF - 2
_VECS_PER_ROW = _DIM // _LANES


def _sc_scale_copy(src_hbm, out_hbm, *scratch):
    bufs = scratch[:_NBUF]
    rsems = scratch[_NBUF:2 * _NBUF]
    wsems = scratch[2 * _NBUF:]
    wid = lax.axis_index("c") * _NS + lax.axis_index("s")
    base = wid * _ROWS_PER_W

    def rows(k):
        return pl.ds(base + k * _CHUNK_ROWS, _CHUNK_ROWS)

    rd = [None] * _NCHUNK
    wr = [None] * _NCHUNK
    for k in range(_AHEAD):
        rd[k] = pltpu.async_copy(src_hbm.at[rows(k)], bufs[k % _NBUF], rsems[k % _NBUF])
    for k in range(_NCHUNK):
        par = k % _NBUF
        rd[k].wait()
        buf = bufs[par]

        @plsc.parallel_loop(0, _CHUNK_ROWS * _VECS_PER_ROW, 1, unroll=8)
        def _scale(i):
            r = i // _VECS_PER_ROW
            c = (i % _VECS_PER_ROW) * _LANES
            buf[r, pl.ds(c, _LANES)] = buf[r, pl.ds(c, _LANES)] * _SCALE

        wr[k] = pltpu.async_copy(buf, out_hbm.at[rows(k)], wsems[par])
        nk = k + _AHEAD
        if nk < _NCHUNK:
            if nk >= _NBUF:
                wr[nk - _NBUF].wait()
            rd[nk] = pltpu.async_copy(
                src_hbm.at[rows(nk)], bufs[nk % _NBUF], rsems[nk % _NBUF])
    for k in range(_NCHUNK - _NBUF, _NCHUNK):
        wr[k].wait()


def kernel(x, embed):
    seq_len = x.shape[1]
    mesh = plsc.VectorSubcoreMesh(
        core_axis_name="c", subcore_axis_name="s",
        num_cores=_NC, num_subcores=_NS,
    )
    run = pl.kernel(
        _sc_scale_copy,
        out_type=jax.ShapeDtypeStruct((seq_len, _DIM), jnp.float32),
        mesh=mesh,
        scratch_types=(
            [pltpu.VMEM((_CHUNK_ROWS, _DIM), jnp.float32)] * _NBUF
            + [pltpu.SemaphoreType.DMA] * (2 * _NBUF)
        ),
    )
    return run(embed)


# speedup vs baseline: 1.0302x; 1.0079x over previous
import jax
import jax.numpy as jnp
from jax import lax
from jax.experimental import pallas as pl
from jax.experimental.pallas import tpu as pltpu
from jax.experimental.pallas import tpu_sc as plsc

_DIM = 1024
_SCALE = _DIM ** (-0.5)

_NC = 2
_NS = 16
_NW = _NC * _NS
_LANES = 16

_ROWS = 8192
_ROWS_PER_W = _ROWS // _NW
_CHUNK_ROWS = 16
_NCHUNK = _ROWS_PER_W // _CHUNK_ROWS
_NBUF = 7
_AHEAD = _NBUF - 2
_VECS_PER_ROW = _DIM // _LANES


def _sc_scale_copy(src_hbm, out_hbm, *scratch):
    bufs = scratch[:_NBUF]
    rsems = scratch[_NBUF:2 * _NBUF]
    wsems = scratch[2 * _NBUF:]
    wid = lax.axis_index("s") * _NC + lax.axis_index("c")
    base = wid * _ROWS_PER_W

    def rows(k):
        return pl.ds(base + k * _CHUNK_ROWS, _CHUNK_ROWS)

    rd = [None] * _NCHUNK
    wr = [None] * _NCHUNK
    for k in range(_AHEAD):
        rd[k] = pltpu.async_copy(src_hbm.at[rows(k)], bufs[k % _NBUF], rsems[k % _NBUF])
    for k in range(_NCHUNK):
        par = k % _NBUF
        rd[k].wait()
        buf = bufs[par]

        @plsc.parallel_loop(0, _CHUNK_ROWS * _VECS_PER_ROW, 1, unroll=8)
        def _scale(i):
            r = i // _VECS_PER_ROW
            c = (i % _VECS_PER_ROW) * _LANES
            buf[r, pl.ds(c, _LANES)] = buf[r, pl.ds(c, _LANES)] * _SCALE

        wr[k] = pltpu.async_copy(buf, out_hbm.at[rows(k)], wsems[par])
        nk = k + _AHEAD
        if nk < _NCHUNK:
            if nk >= _NBUF:
                wr[nk - _NBUF].wait()
            rd[nk] = pltpu.async_copy(
                src_hbm.at[rows(nk)], bufs[nk % _NBUF], rsems[nk % _NBUF])
    for k in range(_NCHUNK - _NBUF, _NCHUNK):
        wr[k].wait()


def kernel(x, embed):
    seq_len = x.shape[1]
    mesh = plsc.VectorSubcoreMesh(
        core_axis_name="c", subcore_axis_name="s",
        num_cores=_NC, num_subcores=_NS,
    )
    run = pl.kernel(
        _sc_scale_copy,
        out_type=jax.ShapeDtypeStruct((seq_len, _DIM), jnp.float32),
        mesh=mesh,
        scratch_types=(
            [pltpu.VMEM((_CHUNK_ROWS, _DIM), jnp.float32)] * _NBUF
            + [pltpu.SemaphoreType.DMA] * (2 * _NBUF)
        ),
    )
    return run(embed)
